# Initial kernel scaffold; baseline (speedup 1.0000x reference)
import jax
import jax.numpy as jnp
from jax.experimental import pallas as pl


def kernel(x, bboxes, categories, img_infos, cls_w1, cls_b1, cls_w2, cls_b2, reg_w1, reg_b1, reg_w2, reg_b2, wh_w1, wh_b1, wh_w2, wh_b2):
    raise NotImplementedError("write your pallas kernel here")


# speedup vs baseline: 4.2857x; 4.2857x over previous
import functools

import jax
import jax.numpy as jnp
from jax import lax
from jax.experimental import pallas as pl

NC = 80
K = 100
H = 128
W = 128
HW = H * W
THR = 0.2


def _head_body(x_ref, w1_ref, b1_ref, w2_ref, b2_ref, o_ref):
    xb = x_ref[0]
    lanes = lax.broadcasted_iota(jnp.int32, (1, HW), 1)
    wpos = lanes % W
    hpos = lanes // W

    acc = jnp.zeros((192, HW), jnp.float32) + b1_ref[...]
    for di in (-1, 0, 1):
        for dj in (-1, 0, 1):
            k = (di + 1) * 3 + (dj + 1)
            s = di * W + dj
            op = xb if s == 0 else jnp.roll(xb, -s, axis=1)
            mask = None
            if di == -1:
                mask = hpos >= 1
            elif di == 1:
                mask = hpos <= H - 2
            if dj == -1:
                m2 = wpos >= 1
                mask = m2 if mask is None else (mask & m2)
            elif dj == 1:
                m2 = wpos <= W - 2
                mask = m2 if mask is None else (mask & m2)
            if mask is not None:
                op = jnp.where(mask, op, 0.0)
            acc = acc + jnp.dot(w1_ref[k], op, preferred_element_type=jnp.float32)

    hid = jnp.maximum(acc, 0.0)
    y2 = jnp.dot(w2_ref[...], hid, preferred_element_type=jnp.float32) + b2_ref[...]

    cls = jax.nn.sigmoid(y2[0:NC])
    rr = jax.nn.sigmoid(y2[NC:NC + 2])
    wh = jnp.exp(y2[NC + 2:NC + 4])

    left = jnp.where(wpos >= 1, jnp.roll(cls, 1, axis=1), -1.0)
    right = jnp.where(wpos <= W - 2, jnp.roll(cls, -1, axis=1), -1.0)
    mw = jnp.maximum(cls, jnp.maximum(left, right))
    up = jnp.where(hpos >= 1, jnp.roll(mw, W, axis=1), -1.0)
    down = jnp.where(hpos <= H - 2, jnp.roll(mw, -W, axis=1), -1.0)
    pooled = jnp.maximum(mw, jnp.maximum(up, down))

    peak = pooled == cls
    msk = jnp.where(peak, cls, 0.0)
    scores = jnp.max(msk, axis=0, keepdims=True)
    rowi = lax.broadcasted_iota(jnp.int32, (NC, HW), 0)
    cats = jnp.min(jnp.where(msk == scores, rowi, NC + 1), axis=0, keepdims=True)

    o_ref[0] = jnp.concatenate(
        [scores, cats.astype(jnp.float32), rr, wh, jnp.zeros((2, HW), jnp.float32)], axis=0
    )


def _cumsum_lanes(c):
    li = lax.broadcasted_iota(jnp.int32, c.shape, 1)
    sh = 1
    while sh < c.shape[1]:
        c = c + jnp.where(li >= sh, jnp.roll(c, sh, axis=1), 0)
        sh *= 2
    return c


def _cumsum_rows(c):
    ri = lax.broadcasted_iota(jnp.int32, c.shape, 0)
    sh = 1
    while sh < c.shape[0]:
        c = c + jnp.where(ri >= sh, jnp.roll(c, sh, axis=0), 0)
        sh *= 2
    return c


def _topk_body(d_ref, o_ref):
    B = 8
    S = d_ref[:, 0, :]
    Si = lax.bitcast_convert_type(S, jnp.int32)

    lo = jnp.zeros((B, 1), jnp.int32)
    hi = jnp.full((B, 1), 0x3F800001, jnp.int32)
    for _ in range(31):
        mid = (lo + hi) // 2
        cnt = jnp.sum((Si >= mid).astype(jnp.int32), axis=1, keepdims=True)
        ge = cnt >= K
        lo = jnp.where(ge, jnp.maximum(mid, lo), lo)
        hi = jnp.where(ge, hi, jnp.minimum(mid, hi))
    m = (Si >= lo).astype(jnp.int32)

    lane_f = lax.broadcasted_iota(jnp.int32, (1, HW), 1).astype(jnp.float32)
    slot_io = lax.broadcasted_iota(jnp.int32, (128, HW), 0)
    cbs = []
    for b in range(B):
        mrow = m[b:b + 1]
        slot = _cumsum_lanes(mrow) - mrow
        A2 = ((slot == slot_io) & (mrow > 0)).astype(jnp.float32)
        Vb = jnp.concatenate([d_ref[b, 0:6, :], lane_f], axis=0)
        cb = lax.dot_general(Vb, A2, (((1,), (1,)), ((), ())),
                             preferred_element_type=jnp.float32,
                             precision=lax.Precision.HIGHEST)
        cbs.append(cb[None])
    C = jnp.concatenate(cbs, axis=0)

    sC = C[:, 0, :]
    catC = C[:, 1, :]
    rxC = C[:, 2, :]
    ryC = C[:, 3, :]
    wxC = C[:, 4, :]
    wyC = C[:, 5, :]
    idxC = C[:, 6, :]

    jio = lax.broadcasted_iota(jnp.int32, (B, 128), 1)
    acc_s = jnp.zeros((B, 128), jnp.float32)
    acc_cat = jnp.zeros((B, 128), jnp.float32)
    acc_rx = jnp.zeros((B, 128), jnp.float32)
    acc_ry = jnp.zeros((B, 128), jnp.float32)
    acc_wx = jnp.zeros((B, 128), jnp.float32)
    acc_wy = jnp.zeros((B, 128), jnp.float32)
    acc_idx = jnp.zeros((B, 128), jnp.float32)
    for j in range(K):
        mx = jnp.max(sC, axis=1, keepdims=True)
        sel = jnp.min(jnp.where(sC == mx, jio, 1 << 30), axis=1, keepdims=True)
        oh = jio == sel
        colm = jio == j
        acc_s = jnp.where(colm, mx, acc_s)
        val = jnp.sum(jnp.where(oh, catC, 0.0), axis=1, keepdims=True)
        acc_cat = jnp.where(colm, val, acc_cat)
        val = jnp.sum(jnp.where(oh, rxC, 0.0), axis=1, keepdims=True)
        acc_rx = jnp.where(colm, val, acc_rx)
        val = jnp.sum(jnp.where(oh, ryC, 0.0), axis=1, keepdims=True)
        acc_ry = jnp.where(colm, val, acc_ry)
        val = jnp.sum(jnp.where(oh, wxC, 0.0), axis=1, keepdims=True)
        acc_wx = jnp.where(colm, val, acc_wx)
        val = jnp.sum(jnp.where(oh, wyC, 0.0), axis=1, keepdims=True)
        acc_wy = jnp.where(colm, val, acc_wy)
        val = jnp.sum(jnp.where(oh, idxC, 0.0), axis=1, keepdims=True)
        acc_idx = jnp.where(colm, val, acc_idx)
        sC = jnp.where(oh, -1.0, sC)

    idxi = acc_idx.astype(jnp.int32)
    gx = (idxi % W).astype(jnp.float32)
    gy = (idxi // W).astype(jnp.float32)
    cx = gx + acc_rx
    cy = gy + acc_ry
    o_ref[:, 0, :] = (cx - acc_wx * 0.5) * 4.0
    o_ref[:, 1, :] = (cy - acc_wy * 0.5) * 4.0
    o_ref[:, 2, :] = (cx + acc_wx * 0.5) * 4.0
    o_ref[:, 3, :] = (cy + acc_wy * 0.5) * 4.0
    o_ref[:, 4, :] = acc_cat
    o_ref[:, 5, :] = acc_s
    o_ref[:, 6, :] = (acc_s > THR).astype(jnp.float32)
    o_ref[:, 7, :] = acc_idx


@functools.partial(jax.jit, static_argnames=("interpret",))
def _run(x, w1r, b1c, w2bd, b2c, interpret=False):
    B = x.shape[0]
    x3 = x.reshape(B, 96, HW)
    heads = pl.pallas_call(
        _head_body,
        grid=(B,),
        in_specs=[
            pl.BlockSpec((1, 96, HW), lambda b: (b, 0, 0)),
            pl.BlockSpec((9, 192, 96), lambda b: (0, 0, 0)),
            pl.BlockSpec((192, 1), lambda b: (0, 0)),
            pl.BlockSpec((96, 192), lambda b: (0, 0)),
            pl.BlockSpec((96, 1), lambda b: (0, 0)),
        ],
        out_specs=pl.BlockSpec((1, 8, HW), lambda b: (b, 0, 0)),
        out_shape=jax.ShapeDtypeStruct((B, 8, HW), jnp.float32),
        interpret=interpret,
    )(x3, w1r, b1c, w2bd, b2c)

    o = pl.pallas_call(
        _topk_body,
        out_shape=jax.ShapeDtypeStruct((B, 8, 128), jnp.float32),
        interpret=interpret,
    )(heads)

    bb = jnp.stack([o[:, 0, :K], o[:, 1, :K], o[:, 2, :K], o[:, 3, :K]], axis=-1)
    cats_k = o[:, 4, :K].astype(jnp.int32)
    topk_scores = o[:, 5, :K]
    valid = o[:, 6, :K] > 0.5
    return bb, cats_k, topk_scores, valid


def kernel(x, bboxes, categories, img_infos, cls_w1, cls_b1, cls_w2, cls_b2,
           reg_w1, reg_b1, reg_w2, reg_b2, wh_w1, wh_b1, wh_w2, wh_b2):
    w1cat = jnp.concatenate([cls_w1, reg_w1, wh_w1], axis=0)
    w1r = jnp.transpose(w1cat, (2, 3, 0, 1)).reshape(9, 192, 96)
    b1c = jnp.concatenate([cls_b1, reg_b1, wh_b1])[:, None]

    w2bd = jnp.zeros((96, 192), jnp.float32)
    w2bd = w2bd.at[0:NC, 0:64].set(cls_w2.reshape(NC, 64))
    w2bd = w2bd.at[NC:NC + 2, 64:128].set(reg_w2.reshape(2, 64))
    w2bd = w2bd.at[NC + 2:NC + 4, 128:192].set(wh_w2.reshape(2, 64))
    b2c = jnp.zeros((96,), jnp.float32)
    b2c = b2c.at[0:NC].set(cls_b2).at[NC:NC + 2].set(reg_b2).at[NC + 2:NC + 4].set(wh_b2)
    b2c = b2c[:, None]

    return _run(x, w1r, b1c, w2bd, b2c)


# speedup vs baseline: 4.5069x; 1.0516x over previous
import functools

import jax
import jax.numpy as jnp
from jax import lax
from jax.experimental import pallas as pl
from jax.experimental.pallas import tpu as pltpu

NC = 80
K = 100
H = 128
W = 128
HW = H * W
THR = 0.2


def _head_body(x_ref, w1_ref, b1_ref, w2_ref, b2_ref, o_ref, x3_ref, acc_ref):
    xb = x_ref[0]
    lanes = lax.broadcasted_iota(jnp.int32, (1, HW), 1)
    wpos = lanes % W
    hpos = lanes // W

    x3_ref[0:96] = jnp.where(wpos >= 1, jnp.roll(xb, 1, axis=1), 0.0)
    x3_ref[96:192] = xb
    x3_ref[192:288] = jnp.where(wpos <= W - 2, jnp.roll(xb, -1, axis=1), 0.0)

    acc_ref[...] = jnp.dot(w1_ref[1], x3_ref[...],
                           preferred_element_type=jnp.float32) + b1_ref[...]
    acc_ref[:, W:] = acc_ref[:, W:] + jnp.dot(
        w1_ref[0], x3_ref[:, :HW - W], preferred_element_type=jnp.float32)
    acc_ref[:, :HW - W] = acc_ref[:, :HW - W] + jnp.dot(
        w1_ref[2], x3_ref[:, W:], preferred_element_type=jnp.float32)

    hid = jnp.maximum(acc_ref[...], 0.0)
    y2 = jnp.dot(w2_ref[...], hid, preferred_element_type=jnp.float32) + b2_ref[...]

    cls = jax.nn.sigmoid(y2[0:NC])
    rr = jax.nn.sigmoid(y2[NC:NC + 2])
    wh = jnp.exp(y2[NC + 2:NC + 4])

    left = jnp.where(wpos >= 1, jnp.roll(cls, 1, axis=1), -1.0)
    right = jnp.where(wpos <= W - 2, jnp.roll(cls, -1, axis=1), -1.0)
    mw = jnp.maximum(cls, jnp.maximum(left, right))
    up = jnp.where(hpos >= 1, jnp.roll(mw, W, axis=1), -1.0)
    down = jnp.where(hpos <= H - 2, jnp.roll(mw, -W, axis=1), -1.0)
    pooled = jnp.maximum(mw, jnp.maximum(up, down))

    peak = pooled == cls
    msk = jnp.where(peak, cls, 0.0)
    scores = jnp.max(msk, axis=0, keepdims=True)
    rowi = lax.broadcasted_iota(jnp.int32, (NC, HW), 0)
    cats = jnp.min(jnp.where(msk == scores, rowi, NC + 1), axis=0, keepdims=True)

    o_ref[0] = jnp.concatenate(
        [scores, cats.astype(jnp.float32), rr, wh, jnp.zeros((2, HW), jnp.float32)], axis=0
    )


def _cumsum_lanes(c):
    li = lax.broadcasted_iota(jnp.int32, c.shape, 1)
    sh = 1
    while sh < c.shape[1]:
        c = c + jnp.where(li >= sh, jnp.roll(c, sh, axis=1), 0)
        sh *= 2
    return c


def _cumsum_rows(c):
    ri = lax.broadcasted_iota(jnp.int32, c.shape, 0)
    sh = 1
    while sh < c.shape[0]:
        c = c + jnp.where(ri >= sh, jnp.roll(c, sh, axis=0), 0)
        sh *= 2
    return c


def _topk_body(d_ref, o_ref):
    B = 8
    S = d_ref[:, 0, :]
    Si = lax.bitcast_convert_type(S, jnp.int32)

    lo = jnp.zeros((B, 1), jnp.int32)
    hi = jnp.full((B, 1), 0x3F800001, jnp.int32)
    for _ in range(31):
        mid = (lo + hi) // 2
        cnt = jnp.sum((Si >= mid).astype(jnp.int32), axis=1, keepdims=True)
        ge = cnt >= K
        lo = jnp.where(ge, jnp.maximum(mid, lo), lo)
        hi = jnp.where(ge, hi, jnp.minimum(mid, hi))
    m = (Si >= lo).astype(jnp.int32)

    lane_f = lax.broadcasted_iota(jnp.int32, (1, HW), 1).astype(jnp.float32)
    slot_io = lax.broadcasted_iota(jnp.int32, (128, HW), 0)
    cbs = []
    for b in range(B):
        mrow = m[b:b + 1]
        slot = _cumsum_lanes(mrow) - mrow
        A2 = ((slot == slot_io) & (mrow > 0)).astype(jnp.float32)
        Vb = jnp.concatenate([d_ref[b, 0:6, :], lane_f], axis=0)
        cb = lax.dot_general(Vb, A2, (((1,), (1,)), ((), ())),
                             preferred_element_type=jnp.float32,
                             precision=lax.Precision.HIGHEST)
        cbs.append(cb[None])
    C = jnp.concatenate(cbs, axis=0)

    sC = C[:, 0, :]
    catC = C[:, 1, :]
    rxC = C[:, 2, :]
    ryC = C[:, 3, :]
    wxC = C[:, 4, :]
    wyC = C[:, 5, :]
    idxC = C[:, 6, :]

    jio = lax.broadcasted_iota(jnp.int32, (B, 128), 1)
    acc_s = jnp.zeros((B, 128), jnp.float32)
    acc_cat = jnp.zeros((B, 128), jnp.float32)
    acc_rx = jnp.zeros((B, 128), jnp.float32)
    acc_ry = jnp.zeros((B, 128), jnp.float32)
    acc_wx = jnp.zeros((B, 128), jnp.float32)
    acc_wy = jnp.zeros((B, 128), jnp.float32)
    acc_idx = jnp.zeros((B, 128), jnp.float32)
    for j in range(K):
        mx = jnp.max(sC, axis=1, keepdims=True)
        sel = jnp.min(jnp.where(sC == mx, jio, 1 << 30), axis=1, keepdims=True)
        oh = jio == sel
        colm = jio == j
        acc_s = jnp.where(colm, mx, acc_s)
        val = jnp.sum(jnp.where(oh, catC, 0.0), axis=1, keepdims=True)
        acc_cat = jnp.where(colm, val, acc_cat)
        val = jnp.sum(jnp.where(oh, rxC, 0.0), axis=1, keepdims=True)
        acc_rx = jnp.where(colm, val, acc_rx)
        val = jnp.sum(jnp.where(oh, ryC, 0.0), axis=1, keepdims=True)
        acc_ry = jnp.where(colm, val, acc_ry)
        val = jnp.sum(jnp.where(oh, wxC, 0.0), axis=1, keepdims=True)
        acc_wx = jnp.where(colm, val, acc_wx)
        val = jnp.sum(jnp.where(oh, wyC, 0.0), axis=1, keepdims=True)
        acc_wy = jnp.where(colm, val, acc_wy)
        val = jnp.sum(jnp.where(oh, idxC, 0.0), axis=1, keepdims=True)
        acc_idx = jnp.where(colm, val, acc_idx)
        sC = jnp.where(oh, -1.0, sC)

    idxi = acc_idx.astype(jnp.int32)
    gx = (idxi % W).astype(jnp.float32)
    gy = (idxi // W).astype(jnp.float32)
    cx = gx + acc_rx
    cy = gy + acc_ry
    o_ref[:, 0, :] = (cx - acc_wx * 0.5) * 4.0
    o_ref[:, 1, :] = (cy - acc_wy * 0.5) * 4.0
    o_ref[:, 2, :] = (cx + acc_wx * 0.5) * 4.0
    o_ref[:, 3, :] = (cy + acc_wy * 0.5) * 4.0
    o_ref[:, 4, :] = acc_cat
    o_ref[:, 5, :] = acc_s
    o_ref[:, 6, :] = (acc_s > THR).astype(jnp.float32)
    o_ref[:, 7, :] = acc_idx


@functools.partial(jax.jit, static_argnames=("interpret",))
def _run(x, w1r, b1c, w2bd, b2c, interpret=False):
    B = x.shape[0]
    x3 = x.reshape(B, 96, HW)
    heads = pl.pallas_call(
        _head_body,
        grid=(B,),
        in_specs=[
            pl.BlockSpec((1, 96, HW), lambda b: (b, 0, 0)),
            pl.BlockSpec((3, 192, 288), lambda b: (0, 0, 0)),
            pl.BlockSpec((192, 1), lambda b: (0, 0)),
            pl.BlockSpec((96, 192), lambda b: (0, 0)),
            pl.BlockSpec((96, 1), lambda b: (0, 0)),
        ],
        out_specs=pl.BlockSpec((1, 8, HW), lambda b: (b, 0, 0)),
        out_shape=jax.ShapeDtypeStruct((B, 8, HW), jnp.float32),
        scratch_shapes=[
            pltpu.VMEM((288, HW), jnp.float32),
            pltpu.VMEM((192, HW), jnp.float32),
        ],
        interpret=interpret,
    )(x3, w1r, b1c, w2bd, b2c)

    o = pl.pallas_call(
        _topk_body,
        out_shape=jax.ShapeDtypeStruct((B, 8, 128), jnp.float32),
        interpret=interpret,
    )(heads)

    bb = jnp.stack([o[:, 0, :K], o[:, 1, :K], o[:, 2, :K], o[:, 3, :K]], axis=-1)
    cats_k = o[:, 4, :K].astype(jnp.int32)
    topk_scores = o[:, 5, :K]
    valid = o[:, 6, :K] > 0.5
    return bb, cats_k, topk_scores, valid


def kernel(x, bboxes, categories, img_infos, cls_w1, cls_b1, cls_w2, cls_b2,
           reg_w1, reg_b1, reg_w2, reg_b2, wh_w1, wh_b1, wh_w2, wh_b2):
    w1cat = jnp.concatenate([cls_w1, reg_w1, wh_w1], axis=0)
    w1r = jnp.transpose(w1cat, (2, 0, 3, 1)).reshape(3, 192, 288)
    b1c = jnp.concatenate([cls_b1, reg_b1, wh_b1])[:, None]

    w2bd = jnp.zeros((96, 192), jnp.float32)
    w2bd = w2bd.at[0:NC, 0:64].set(cls_w2.reshape(NC, 64))
    w2bd = w2bd.at[NC:NC + 2, 64:128].set(reg_w2.reshape(2, 64))
    w2bd = w2bd.at[NC + 2:NC + 4, 128:192].set(wh_w2.reshape(2, 64))
    b2c = jnp.zeros((96,), jnp.float32)
    b2c = b2c.at[0:NC].set(cls_b2).at[NC:NC + 2].set(reg_b2).at[NC + 2:NC + 4].set(wh_b2)
    b2c = b2c[:, None]

    return _run(x, w1r, b1c, w2bd, b2c)


# speedup vs baseline: 4.8756x; 1.0818x over previous
import functools

import jax
import jax.numpy as jnp
from jax import lax
from jax.experimental import pallas as pl
from jax.experimental.pallas import tpu as pltpu

NC = 80
K = 100
H = 128
W = 128
HW = H * W
THR = 0.2


def _head_body(x_ref, w1_ref, b1_ref, w2_ref, b2_ref, o_ref, x3_ref, acc_ref):
    xb = x_ref[0]
    lanes = lax.broadcasted_iota(jnp.int32, (1, HW), 1)
    wpos = lanes % W
    hpos = lanes // W

    x3_ref[0:96] = jnp.where(wpos >= 1, jnp.roll(xb, 1, axis=1), 0.0)
    x3_ref[96:192] = xb
    x3_ref[192:288] = jnp.where(wpos <= W - 2, jnp.roll(xb, -1, axis=1), 0.0)

    acc_ref[...] = jnp.dot(w1_ref[1], x3_ref[...],
                           preferred_element_type=jnp.float32) + b1_ref[...]
    acc_ref[:, W:] = acc_ref[:, W:] + jnp.dot(
        w1_ref[0], x3_ref[:, :HW - W], preferred_element_type=jnp.float32)
    acc_ref[:, :HW - W] = acc_ref[:, :HW - W] + jnp.dot(
        w1_ref[2], x3_ref[:, W:], preferred_element_type=jnp.float32)

    hid = jnp.maximum(acc_ref[...], 0.0)
    y2 = jnp.dot(w2_ref[...], hid, preferred_element_type=jnp.float32) + b2_ref[...]

    cls = jax.nn.sigmoid(y2[0:NC])
    rr = jax.nn.sigmoid(y2[NC:NC + 2])
    wh = jnp.exp(y2[NC + 2:NC + 4])

    left = jnp.where(wpos >= 1, jnp.roll(cls, 1, axis=1), -1.0)
    right = jnp.where(wpos <= W - 2, jnp.roll(cls, -1, axis=1), -1.0)
    mw = jnp.maximum(cls, jnp.maximum(left, right))
    up = jnp.where(hpos >= 1, jnp.roll(mw, W, axis=1), -1.0)
    down = jnp.where(hpos <= H - 2, jnp.roll(mw, -W, axis=1), -1.0)
    pooled = jnp.maximum(mw, jnp.maximum(up, down))

    peak = pooled == cls
    msk = jnp.where(peak, cls, 0.0)
    scores = jnp.max(msk, axis=0, keepdims=True)
    rowi = lax.broadcasted_iota(jnp.int32, (NC, HW), 0)
    cats = jnp.min(jnp.where(msk == scores, rowi, NC + 1), axis=0, keepdims=True)

    lane_f = lanes.astype(jnp.float32)
    o_ref[0] = jnp.concatenate(
        [scores, cats.astype(jnp.float32), rr, wh, lane_f, jnp.zeros((1, HW), jnp.float32)],
        axis=0,
    )


def _cumsum_lanes(c):
    li = lax.broadcasted_iota(jnp.int32, c.shape, 1)
    sh = 1
    while sh < c.shape[1]:
        c = c + jnp.where(li >= sh, jnp.roll(c, sh, axis=1), 0)
        sh *= 2
    return c


def _cumsum_rows(c):
    ri = lax.broadcasted_iota(jnp.int32, c.shape, 0)
    sh = 1
    while sh < c.shape[0]:
        c = c + jnp.where(ri >= sh, jnp.roll(c, sh, axis=0), 0)
        sh *= 2
    return c


def _topk_body(d_ref, o_ref):
    B = 8
    S = d_ref[:, 0, :]
    Si = lax.bitcast_convert_type(S, jnp.int32)

    lo = jnp.zeros((B, 1), jnp.int32)
    hi = jnp.full((B, 1), 0x3F800001, jnp.int32)
    for _ in range(31):
        mid = (lo + hi) // 2
        cnt = jnp.sum((Si >= mid).astype(jnp.int32), axis=1, keepdims=True)
        ge = cnt >= K
        lo = jnp.where(ge, jnp.maximum(mid, lo), lo)
        hi = jnp.where(ge, hi, jnp.minimum(mid, hi))
    m = (Si >= lo).astype(jnp.int32)

    slot_all = _cumsum_lanes(m) - m
    slot_m = jnp.where(m > 0, slot_all, -1)

    slot_io = lax.broadcasted_iota(jnp.int32, (128, HW), 0)
    cbs = []
    for b in range(B):
        A2 = (slot_m[b:b + 1] == slot_io).astype(jnp.float32)
        Vb = d_ref[b, 0:7, :]
        cb = lax.dot_general(Vb, A2, (((1,), (1,)), ((), ())),
                             preferred_element_type=jnp.float32,
                             precision=lax.Precision.HIGHEST)
        cbs.append(cb[None])
    C = jnp.concatenate(cbs, axis=0)

    sC = C[:, 0, :]
    catC = C[:, 1, :]
    rxC = C[:, 2, :]
    ryC = C[:, 3, :]
    wxC = C[:, 4, :]
    wyC = C[:, 5, :]
    idxC = C[:, 6, :]

    jio = lax.broadcasted_iota(jnp.int32, (B, 128), 1)
    acc_s = jnp.zeros((B, 128), jnp.float32)
    acc_cat = jnp.zeros((B, 128), jnp.float32)
    acc_rx = jnp.zeros((B, 128), jnp.float32)
    acc_ry = jnp.zeros((B, 128), jnp.float32)
    acc_wx = jnp.zeros((B, 128), jnp.float32)
    acc_wy = jnp.zeros((B, 128), jnp.float32)
    acc_idx = jnp.zeros((B, 128), jnp.float32)
    for j in range(K):
        mx = jnp.max(sC, axis=1, keepdims=True)
        sel = jnp.min(jnp.where(sC == mx, jio, 1 << 30), axis=1, keepdims=True)
        oh = jio == sel
        colm = jio == j
        acc_s = jnp.where(colm, mx, acc_s)
        val = jnp.sum(jnp.where(oh, catC, 0.0), axis=1, keepdims=True)
        acc_cat = jnp.where(colm, val, acc_cat)
        val = jnp.sum(jnp.where(oh, rxC, 0.0), axis=1, keepdims=True)
        acc_rx = jnp.where(colm, val, acc_rx)
        val = jnp.sum(jnp.where(oh, ryC, 0.0), axis=1, keepdims=True)
        acc_ry = jnp.where(colm, val, acc_ry)
        val = jnp.sum(jnp.where(oh, wxC, 0.0), axis=1, keepdims=True)
        acc_wx = jnp.where(colm, val, acc_wx)
        val = jnp.sum(jnp.where(oh, wyC, 0.0), axis=1, keepdims=True)
        acc_wy = jnp.where(colm, val, acc_wy)
        val = jnp.sum(jnp.where(oh, idxC, 0.0), axis=1, keepdims=True)
        acc_idx = jnp.where(colm, val, acc_idx)
        sC = jnp.where(oh, -1.0, sC)

    idxi = acc_idx.astype(jnp.int32)
    gx = (idxi % W).astype(jnp.float32)
    gy = (idxi // W).astype(jnp.float32)
    cx = gx + acc_rx
    cy = gy + acc_ry
    o_ref[:, 0, :] = (cx - acc_wx * 0.5) * 4.0
    o_ref[:, 1, :] = (cy - acc_wy * 0.5) * 4.0
    o_ref[:, 2, :] = (cx + acc_wx * 0.5) * 4.0
    o_ref[:, 3, :] = (cy + acc_wy * 0.5) * 4.0
    o_ref[:, 4, :] = acc_cat
    o_ref[:, 5, :] = acc_s
    o_ref[:, 6, :] = (acc_s > THR).astype(jnp.float32)
    o_ref[:, 7, :] = acc_idx


@functools.partial(jax.jit, static_argnames=("interpret",))
def _run(x, w1r, b1c, w2bd, b2c, interpret=False):
    B = x.shape[0]
    x3 = x.reshape(B, 96, HW)
    heads = pl.pallas_call(
        _head_body,
        grid=(B,),
        in_specs=[
            pl.BlockSpec((1, 96, HW), lambda b: (b, 0, 0)),
            pl.BlockSpec((3, 192, 288), lambda b: (0, 0, 0)),
            pl.BlockSpec((192, 1), lambda b: (0, 0)),
            pl.BlockSpec((96, 192), lambda b: (0, 0)),
            pl.BlockSpec((96, 1), lambda b: (0, 0)),
        ],
        out_specs=pl.BlockSpec((1, 8, HW), lambda b: (b, 0, 0)),
        out_shape=jax.ShapeDtypeStruct((B, 8, HW), jnp.float32),
        scratch_shapes=[
            pltpu.VMEM((288, HW), jnp.float32),
            pltpu.VMEM((192, HW), jnp.float32),
        ],
        interpret=interpret,
    )(x3, w1r, b1c, w2bd, b2c)

    o = pl.pallas_call(
        _topk_body,
        out_shape=jax.ShapeDtypeStruct((B, 8, 128), jnp.float32),
        interpret=interpret,
    )(heads)

    bb = jnp.stack([o[:, 0, :K], o[:, 1, :K], o[:, 2, :K], o[:, 3, :K]], axis=-1)
    cats_k = o[:, 4, :K].astype(jnp.int32)
    topk_scores = o[:, 5, :K]
    valid = o[:, 6, :K] > 0.5
    return bb, cats_k, topk_scores, valid


def kernel(x, bboxes, categories, img_infos, cls_w1, cls_b1, cls_w2, cls_b2,
           reg_w1, reg_b1, reg_w2, reg_b2, wh_w1, wh_b1, wh_w2, wh_b2):
    w1cat = jnp.concatenate([cls_w1, reg_w1, wh_w1], axis=0)
    w1r = jnp.transpose(w1cat, (2, 0, 3, 1)).reshape(3, 192, 288)
    b1c = jnp.concatenate([cls_b1, reg_b1, wh_b1])[:, None]

    w2bd = jnp.zeros((96, 192), jnp.float32)
    w2bd = w2bd.at[0:NC, 0:64].set(cls_w2.reshape(NC, 64))
    w2bd = w2bd.at[NC:NC + 2, 64:128].set(reg_w2.reshape(2, 64))
    w2bd = w2bd.at[NC + 2:NC + 4, 128:192].set(wh_w2.reshape(2, 64))
    b2c = jnp.zeros((96,), jnp.float32)
    b2c = b2c.at[0:NC].set(cls_b2).at[NC:NC + 2].set(reg_b2).at[NC + 2:NC + 4].set(wh_b2)
    b2c = b2c[:, None]

    return _run(x, w1r, b1c, w2bd, b2c)


# speedup vs baseline: 5.3988x; 1.1073x over previous
import functools

import jax
import jax.numpy as jnp
from jax import lax
from jax.experimental import pallas as pl
from jax.experimental.pallas import tpu as pltpu

NC = 80
K = 100
H = 128
W = 128
HW = H * W
THR = 0.2


def _head_body(x_ref, w1_ref, b1_ref, w2_ref, b2_ref, o_ref, x3_ref, acc_ref):
    xb = x_ref[0]
    lanes = lax.broadcasted_iota(jnp.int32, (1, HW), 1)
    wpos = lanes % W
    hpos = lanes // W

    xh = xb.astype(jnp.bfloat16)
    zb = jnp.zeros((), jnp.bfloat16)
    for kh in range(3):
        for kw in range(3):
            s = (kh - 1) * W + (kw - 1)
            op = xh if s == 0 else jnp.roll(xh, -s, axis=1)
            if kw == 0:
                op = jnp.where(wpos >= 1, op, zb)
            elif kw == 2:
                op = jnp.where(wpos <= W - 2, op, zb)
            k = kh * 3 + kw
            x3_ref[k * 96:(k + 1) * 96] = op

    acc_ref[...] = jnp.dot(w1_ref[...], x3_ref[...],
                           preferred_element_type=jnp.float32)
    acc_ref[:, 0:W] = jnp.dot(w1_ref[:, 288:864], x3_ref[288:864, 0:W],
                              preferred_element_type=jnp.float32)
    acc_ref[:, HW - W:] = jnp.dot(w1_ref[:, 0:576], x3_ref[0:576, HW - W:],
                                  preferred_element_type=jnp.float32)

    hid = jnp.maximum(acc_ref[...] + b1_ref[...], 0.0).astype(jnp.bfloat16)
    y2 = jnp.dot(w2_ref[...], hid, preferred_element_type=jnp.float32) + b2_ref[...]

    cls = jax.nn.sigmoid(y2[0:NC])
    rr = jax.nn.sigmoid(y2[NC:NC + 2])
    wh = jnp.exp(y2[NC + 2:NC + 4])

    left = jnp.where(wpos >= 1, jnp.roll(cls, 1, axis=1), -1.0)
    right = jnp.where(wpos <= W - 2, jnp.roll(cls, -1, axis=1), -1.0)
    mw = jnp.maximum(cls, jnp.maximum(left, right))
    up = jnp.where(hpos >= 1, jnp.roll(mw, W, axis=1), -1.0)
    down = jnp.where(hpos <= H - 2, jnp.roll(mw, -W, axis=1), -1.0)
    pooled = jnp.maximum(mw, jnp.maximum(up, down))

    peak = pooled == cls
    msk = jnp.where(peak, cls, 0.0)
    scores = jnp.max(msk, axis=0, keepdims=True)
    rowi = lax.broadcasted_iota(jnp.int32, (NC, HW), 0)
    cats = jnp.min(jnp.where(msk == scores, rowi, NC + 1), axis=0, keepdims=True)

    lane_f = lanes.astype(jnp.float32)
    o_ref[0] = jnp.concatenate(
        [scores, cats.astype(jnp.float32), rr, wh, lane_f, jnp.zeros((1, HW), jnp.float32)],
        axis=0,
    )


def _cumsum_lanes(c):
    li = lax.broadcasted_iota(jnp.int32, c.shape, 1)
    sh = 1
    while sh < c.shape[1]:
        c = c + jnp.where(li >= sh, jnp.roll(c, sh, axis=1), 0)
        sh *= 2
    return c


def _cumsum_rows(c):
    ri = lax.broadcasted_iota(jnp.int32, c.shape, 0)
    sh = 1
    while sh < c.shape[0]:
        c = c + jnp.where(ri >= sh, jnp.roll(c, sh, axis=0), 0)
        sh *= 2
    return c


def _topk_body(d_ref, o_ref):
    B = 8
    S = d_ref[:, 0, :]
    Si = lax.bitcast_convert_type(S, jnp.int32)

    lo = jnp.zeros((B, 1), jnp.int32)
    hi = jnp.full((B, 1), 0x3F800001, jnp.int32)
    for _ in range(31):
        mid = (lo + hi) // 2
        cnt = jnp.sum((Si >= mid).astype(jnp.int32), axis=1, keepdims=True)
        ge = cnt >= K
        lo = jnp.where(ge, jnp.maximum(mid, lo), lo)
        hi = jnp.where(ge, hi, jnp.minimum(mid, hi))
    m = (Si >= lo).astype(jnp.int32)

    slot_all = _cumsum_lanes(m) - m
    slot_m = jnp.where(m > 0, slot_all, -1)

    slot_io = lax.broadcasted_iota(jnp.int32, (128, HW), 0)
    cbs = []
    for b in range(B):
        A2 = (slot_m[b:b + 1] == slot_io).astype(jnp.float32)
        Vb = d_ref[b, 0:7, :]
        cb = lax.dot_general(Vb, A2, (((1,), (1,)), ((), ())),
                             preferred_element_type=jnp.float32,
                             precision=lax.Precision.HIGHEST)
        cbs.append(cb[None])
    C = jnp.concatenate(cbs, axis=0)

    sC = C[:, 0, :]
    catC = C[:, 1, :]
    rxC = C[:, 2, :]
    ryC = C[:, 3, :]
    wxC = C[:, 4, :]
    wyC = C[:, 5, :]
    idxC = C[:, 6, :]

    jio = lax.broadcasted_iota(jnp.int32, (B, 128), 1)
    acc_s = jnp.zeros((B, 128), jnp.float32)
    acc_cat = jnp.zeros((B, 128), jnp.float32)
    acc_rx = jnp.zeros((B, 128), jnp.float32)
    acc_ry = jnp.zeros((B, 128), jnp.float32)
    acc_wx = jnp.zeros((B, 128), jnp.float32)
    acc_wy = jnp.zeros((B, 128), jnp.float32)
    acc_idx = jnp.zeros((B, 128), jnp.float32)
    for j in range(K):
        mx = jnp.max(sC, axis=1, keepdims=True)
        sel = jnp.min(jnp.where(sC == mx, jio, 1 << 30), axis=1, keepdims=True)
        oh = jio == sel
        colm = jio == j
        acc_s = jnp.where(colm, mx, acc_s)
        val = jnp.sum(jnp.where(oh, catC, 0.0), axis=1, keepdims=True)
        acc_cat = jnp.where(colm, val, acc_cat)
        val = jnp.sum(jnp.where(oh, rxC, 0.0), axis=1, keepdims=True)
        acc_rx = jnp.where(colm, val, acc_rx)
        val = jnp.sum(jnp.where(oh, ryC, 0.0), axis=1, keepdims=True)
        acc_ry = jnp.where(colm, val, acc_ry)
        val = jnp.sum(jnp.where(oh, wxC, 0.0), axis=1, keepdims=True)
        acc_wx = jnp.where(colm, val, acc_wx)
        val = jnp.sum(jnp.where(oh, wyC, 0.0), axis=1, keepdims=True)
        acc_wy = jnp.where(colm, val, acc_wy)
        val = jnp.sum(jnp.where(oh, idxC, 0.0), axis=1, keepdims=True)
        acc_idx = jnp.where(colm, val, acc_idx)
        sC = jnp.where(oh, -1.0, sC)

    idxi = acc_idx.astype(jnp.int32)
    gx = (idxi % W).astype(jnp.float32)
    gy = (idxi // W).astype(jnp.float32)
    cx = gx + acc_rx
    cy = gy + acc_ry
    o_ref[:, 0, :] = (cx - acc_wx * 0.5) * 4.0
    o_ref[:, 1, :] = (cy - acc_wy * 0.5) * 4.0
    o_ref[:, 2, :] = (cx + acc_wx * 0.5) * 4.0
    o_ref[:, 3, :] = (cy + acc_wy * 0.5) * 4.0
    o_ref[:, 4, :] = acc_cat
    o_ref[:, 5, :] = acc_s
    o_ref[:, 6, :] = (acc_s > THR).astype(jnp.float32)
    o_ref[:, 7, :] = acc_idx


@functools.partial(jax.jit, static_argnames=("interpret",))
def _run(x, w1r, b1c, w2bd, b2c, interpret=False):
    B = x.shape[0]
    x3 = x.reshape(B, 96, HW)
    heads = pl.pallas_call(
        _head_body,
        grid=(B,),
        in_specs=[
            pl.BlockSpec((1, 96, HW), lambda b: (b, 0, 0)),
            pl.BlockSpec((192, 864), lambda b: (0, 0)),
            pl.BlockSpec((192, 1), lambda b: (0, 0)),
            pl.BlockSpec((96, 192), lambda b: (0, 0)),
            pl.BlockSpec((96, 1), lambda b: (0, 0)),
        ],
        out_specs=pl.BlockSpec((1, 8, HW), lambda b: (b, 0, 0)),
        out_shape=jax.ShapeDtypeStruct((B, 8, HW), jnp.float32),
        scratch_shapes=[
            pltpu.VMEM((864, HW), jnp.bfloat16),
            pltpu.VMEM((192, HW), jnp.float32),
        ],
        interpret=interpret,
    )(x3, w1r, b1c, w2bd, b2c)

    o = pl.pallas_call(
        _topk_body,
        out_shape=jax.ShapeDtypeStruct((B, 8, 128), jnp.float32),
        interpret=interpret,
    )(heads)

    bb = jnp.stack([o[:, 0, :K], o[:, 1, :K], o[:, 2, :K], o[:, 3, :K]], axis=-1)
    cats_k = o[:, 4, :K].astype(jnp.int32)
    topk_scores = o[:, 5, :K]
    valid = o[:, 6, :K] > 0.5
    return bb, cats_k, topk_scores, valid


def kernel(x, bboxes, categories, img_infos, cls_w1, cls_b1, cls_w2, cls_b2,
           reg_w1, reg_b1, reg_w2, reg_b2, wh_w1, wh_b1, wh_w2, wh_b2):
    w1cat = jnp.concatenate([cls_w1, reg_w1, wh_w1], axis=0)
    w1r = jnp.transpose(w1cat, (0, 2, 3, 1)).reshape(192, 864).astype(jnp.bfloat16)
    b1c = jnp.concatenate([cls_b1, reg_b1, wh_b1])[:, None]

    w2bd = jnp.zeros((96, 192), jnp.float32)
    w2bd = w2bd.at[0:NC, 0:64].set(cls_w2.reshape(NC, 64))
    w2bd = w2bd.at[NC:NC + 2, 64:128].set(reg_w2.reshape(2, 64))
    w2bd = w2bd.at[NC + 2:NC + 4, 128:192].set(wh_w2.reshape(2, 64))
    w2bd = w2bd.astype(jnp.bfloat16)
    b2c = jnp.zeros((96,), jnp.float32)
    b2c = b2c.at[0:NC].set(cls_b2).at[NC:NC + 2].set(reg_b2).at[NC + 2:NC + 4].set(wh_b2)
    b2c = b2c[:, None]

    return _run(x, w1r, b1c, w2bd, b2c)
